# baseline (speedup 1.0000x reference)
import functools

import jax
import jax.numpy as jnp
from jax import lax
from jax.experimental import pallas as pl
from jax.experimental.pallas import tpu as pltpu

B, S, D, E, K = 4, 8192, 1024, 8, 2
T = B * S


def _router_body(x_ref, w_ref, wout_ref, sel_ref, psum_ref, cnt_ref, aux_ref):
    pid = pl.program_id(0)
    nblocks = pl.num_programs(0)

    @pl.when(pid == 0)
    def _init():
        psum_ref[...] = jnp.zeros_like(psum_ref)
        cnt_ref[...] = jnp.zeros_like(cnt_ref)
        aux_ref[...] = jnp.zeros_like(aux_ref)

    x = x_ref[...]
    w = w_ref[...]
    logits = lax.dot_general(
        x, w, (((1,), (1,)), ((), ())), preferred_element_type=jnp.float32
    )

    m = jnp.max(logits, axis=-1, keepdims=True)
    ex = jnp.exp(logits - m)
    s = jnp.sum(ex, axis=-1, keepdims=True)
    p = ex / s

    iota = lax.broadcasted_iota(jnp.int32, p.shape, 1)
    m1 = jnp.max(p, axis=-1, keepdims=True)
    i1 = jnp.min(jnp.where(p == m1, iota, E), axis=-1, keepdims=True)
    p_m = jnp.where(iota == i1, -1.0, p)
    m2 = jnp.max(p_m, axis=-1, keepdims=True)
    i2 = jnp.min(jnp.where(p_m == m2, iota, E), axis=-1, keepdims=True)

    denom = m1 + m2
    wout_ref[...] = jnp.concatenate([m1 / denom, m2 / denom], axis=1)
    sel_ref[...] = jnp.concatenate([i1, i2], axis=1)

    onehot = (iota == i1).astype(jnp.float32) + (iota == i2).astype(jnp.float32)
    psum_ref[...] += jnp.sum(p, axis=0, keepdims=True)
    cnt_ref[...] += jnp.sum(onehot, axis=0, keepdims=True)

    @pl.when(pid == nblocks - 1)
    def _fin():
        aux_ref[...] = (
            jnp.float32(E)
            * jnp.sum(psum_ref[...] * cnt_ref[...], keepdims=True)
            / jnp.float32(T * T)
        )[:, :1]


@functools.partial(jax.jit, static_argnames=("bt",))
def _router(x, gate_w, bt=2048):
    nb = T // bt
    wout, sel, _, _, aux = pl.pallas_call(
        _router_body,
        grid=(nb,),
        in_specs=[
            pl.BlockSpec((bt, D), lambda i: (i, 0)),
            pl.BlockSpec((E, D), lambda i: (0, 0)),
        ],
        out_specs=[
            pl.BlockSpec((bt, K), lambda i: (i, 0)),
            pl.BlockSpec((bt, K), lambda i: (i, 0)),
            pl.BlockSpec((1, E), lambda i: (0, 0)),
            pl.BlockSpec((1, E), lambda i: (0, 0)),
            pl.BlockSpec((1, 1), lambda i: (0, 0)),
        ],
        out_shape=[
            jax.ShapeDtypeStruct((T, K), jnp.float32),
            jax.ShapeDtypeStruct((T, K), jnp.int32),
            jax.ShapeDtypeStruct((1, E), jnp.float32),
            jax.ShapeDtypeStruct((1, E), jnp.float32),
            jax.ShapeDtypeStruct((1, 1), jnp.float32),
        ],
        compiler_params=pltpu.CompilerParams(
            dimension_semantics=("arbitrary",),
        ),
    )(x, gate_w)
    return wout, sel, aux


def kernel(hidden_states, gate_w):
    x = hidden_states.reshape(T, D)
    wout, sel, aux = _router(x, gate_w)
    routing_weights = wout.reshape(B, S, K, 1)
    selected_experts = sel.reshape(B, S, K)
    return routing_weights, selected_experts, aux.reshape(())


# speedup vs baseline: 1.8268x; 1.8268x over previous
import functools

import jax
import jax.numpy as jnp
from jax import lax
from jax.experimental import pallas as pl
from jax.experimental.pallas import tpu as pltpu

B, S, D, E, K = 4, 8192, 1024, 8, 2
T = B * S


def _router_body(x_ref, w_ref, wout_ref, sel_ref, psum_ref, cnt_ref, aux_ref):
    pid = pl.program_id(0)
    nblocks = pl.num_programs(0)

    @pl.when(pid == 0)
    def _init():
        psum_ref[...] = jnp.zeros_like(psum_ref)
        cnt_ref[...] = jnp.zeros_like(cnt_ref)
        aux_ref[...] = jnp.zeros_like(aux_ref)

    x = x_ref[...]
    w = w_ref[...]
    logits = lax.dot_general(
        w, x, (((1,), (1,)), ((), ())), preferred_element_type=jnp.float32
    )

    m = jnp.max(logits, axis=0, keepdims=True)
    ex = jnp.exp(logits - m)
    s = jnp.sum(ex, axis=0, keepdims=True)
    p = ex / s

    iota = lax.broadcasted_iota(jnp.int32, p.shape, 0)
    m1 = jnp.max(p, axis=0, keepdims=True)
    i1 = jnp.min(jnp.where(p == m1, iota, E), axis=0, keepdims=True)
    p_m = jnp.where(iota == i1, -1.0, p)
    m2 = jnp.max(p_m, axis=0, keepdims=True)
    i2 = jnp.min(jnp.where(p_m == m2, iota, E), axis=0, keepdims=True)

    denom = m1 + m2
    wout_ref[...] = jnp.concatenate([m1 / denom, m2 / denom], axis=0)
    sel_ref[...] = jnp.concatenate([i1, i2], axis=0)

    onehot = (iota == i1).astype(jnp.float32) + (iota == i2).astype(jnp.float32)
    psum_ref[...] += jnp.sum(p, axis=1, keepdims=True)
    cnt_ref[...] += jnp.sum(onehot, axis=1, keepdims=True)

    @pl.when(pid == nblocks - 1)
    def _fin():
        aux_ref[...] = (
            jnp.float32(E)
            * jnp.sum(psum_ref[...] * cnt_ref[...], keepdims=True)
            / jnp.float32(T * T)
        )[:1, :]


@functools.partial(jax.jit, static_argnames=("bt",))
def _router(x, gate_w, bt=4096):
    nb = T // bt
    wout, sel, _, _, aux = pl.pallas_call(
        _router_body,
        grid=(nb,),
        in_specs=[
            pl.BlockSpec((bt, D), lambda i: (i, 0)),
            pl.BlockSpec((E, D), lambda i: (0, 0)),
        ],
        out_specs=[
            pl.BlockSpec((K, bt), lambda i: (0, i)),
            pl.BlockSpec((K, bt), lambda i: (0, i)),
            pl.BlockSpec((E, 1), lambda i: (0, 0)),
            pl.BlockSpec((E, 1), lambda i: (0, 0)),
            pl.BlockSpec((1, 1), lambda i: (0, 0)),
        ],
        out_shape=[
            jax.ShapeDtypeStruct((K, T), jnp.float32),
            jax.ShapeDtypeStruct((K, T), jnp.int32),
            jax.ShapeDtypeStruct((E, 1), jnp.float32),
            jax.ShapeDtypeStruct((E, 1), jnp.float32),
            jax.ShapeDtypeStruct((1, 1), jnp.float32),
        ],
        compiler_params=pltpu.CompilerParams(
            dimension_semantics=("arbitrary",),
        ),
    )(x, gate_w)
    return wout, sel, aux


def kernel(hidden_states, gate_w):
    x = hidden_states.reshape(T, D)
    wout, sel, aux = _router(x, gate_w)
    routing_weights = wout.T.reshape(B, S, K, 1)
    selected_experts = sel.T.reshape(B, S, K)
    return routing_weights, selected_experts, aux.reshape(())
